# Initial kernel scaffold; baseline (speedup 1.0000x reference)
import jax
import jax.numpy as jnp
from jax.experimental import pallas as pl


def kernel(x, prev_h, edge_index, Q_neigh, B):
    raise NotImplementedError("write your pallas kernel here")


# speedup vs baseline: 55.0923x; 55.0923x over previous
import functools

import jax
import jax.numpy as jnp
from jax import lax
from jax.experimental import pallas as pl
from jax.experimental.pallas import tpu as pltpu
from jax.experimental.pallas import tpu_sc as plsc

N = 50000
E = 800000
C = 10
M = 32
NG = 8
F = C * NG
FA = 32
FB = 16

NC = 2
NS = 16
LANES = 16

HALF = N // NC
ROWS = 25088
DUMP = HALF
STRIPE = ROWS // NS
WB_FULL = STRIPE
WB_LAST = HALF - 15 * STRIPE

K = 128
EPT = 50048
E_PAD = NS * EPT
CHUNKS = EPT // K


def _sc_segment_sum(src, dst, ph_a, ph_b, ph_c):
    mesh = plsc.VectorSubcoreMesh(core_axis_name="c", subcore_axis_name="s",
                                  num_cores=NC, num_subcores=NS)

    @functools.partial(
        pl.kernel,
        out_type=(
            jax.ShapeDtypeStruct((N, FA), jnp.float32),
            jax.ShapeDtypeStruct((N, FB), jnp.float32),
            jax.ShapeDtypeStruct((N, FA), jnp.float32),
            jax.ShapeDtypeStruct((N,), jnp.float32),
        ),
        mesh=mesh,
        compiler_params=pltpu.CompilerParams(use_tc_tiling_on_sc=False),
        scratch_types=dict(
            src_v=pltpu.VMEM((K,), jnp.int32),
            dst_v=pltpu.VMEM((K,), jnp.int32),
            idx_v=pltpu.VMEM((K,), jnp.int32),
            ones_v=pltpu.VMEM((K,), jnp.float32),
            rows_a=pltpu.VMEM((K, FA), jnp.float32),
            rows_b=pltpu.VMEM((K, FB), jnp.float32),
            zbuf_a=pltpu.VMEM((224, FA), jnp.float32),
            zbuf_b=pltpu.VMEM((224, FB), jnp.float32),
            zcnt=pltpu.VMEM((STRIPE,), jnp.float32),
            acc_a=pltpu.VMEM_SHARED((ROWS, FA), jnp.float32),
            acc_b=pltpu.VMEM_SHARED((ROWS, FB), jnp.float32),
            cnt=pltpu.VMEM_SHARED((ROWS,), jnp.float32),
            sem=pltpu.SemaphoreType.DMA,
        ),
    )
    def k(src_hbm, dst_hbm, a_hbm, b_hbm, c_hbm, sa_hbm, sb_hbm, sc_hbm,
          cnt_hbm, *, src_v, dst_v, idx_v, ones_v, rows_a, rows_b,
          zbuf_a, zbuf_b, zcnt, acc_a, acc_b, cnt, sem):
        cid = lax.axis_index("c")
        sid = lax.axis_index("s")
        base = cid * HALF
        o_base = cid * HALF + sid * STRIPE

        zero16 = jnp.zeros((LANES,), jnp.float32)
        one16 = jnp.ones((LANES,), jnp.float32)
        for j in range(K // LANES):
            ones_v[pl.ds(j * LANES, LANES)] = one16

        def zrow(i, carry):
            for j in range(FA // LANES):
                zbuf_a[i, pl.ds(j * LANES, LANES)] = zero16
            zbuf_b[i, pl.ds(0, LANES)] = zero16
            return carry
        lax.fori_loop(0, 224, zrow, 0)

        def zc(i, carry):
            zcnt[pl.ds(i * LANES, LANES)] = zero16
            return carry
        lax.fori_loop(0, STRIPE // LANES, zc, 0)

        def zero_acc_a():
            for t in range(STRIPE // 224):
                pltpu.sync_copy(zbuf_a,
                                acc_a.at[pl.ds(sid * STRIPE + t * 224, 224)])

        def edge_loop(with_b):
            def body(j, carry):
                off = sid * EPT + j * K
                pltpu.sync_copy(src_hbm.at[pl.ds(off, K)], src_v)
                pltpu.sync_copy(dst_hbm.at[pl.ds(off, K)], dst_v)
                if with_b:
                    ga = pltpu.async_copy(a_hbm.at[dst_v], rows_a, sem)
                    pltpu.async_copy(b_hbm.at[dst_v], rows_b, sem).wait()
                else:
                    ga = pltpu.async_copy(c_hbm.at[dst_v], rows_a, sem)
                ga.wait()
                for j2 in range(K // LANES):
                    s16 = src_v[pl.ds(j2 * LANES, LANES)]
                    loc = s16 - base
                    ok = (loc >= 0) & (loc < HALF)
                    idx_v[pl.ds(j2 * LANES, LANES)] = jnp.where(ok, loc, DUMP)
                pltpu.sync_copy(rows_a, acc_a.at[idx_v], add=True)
                if with_b:
                    pltpu.sync_copy(rows_b, acc_b.at[idx_v], add=True)
                    pltpu.sync_copy(ones_v, cnt.at[idx_v], add=True)
                return carry
            lax.fori_loop(0, CHUNKS, body, 0)

        def writeback(acc_ref, out_hbm, width_is_b=False):
            @pl.when(sid < NS - 1)
            def _():
                pltpu.sync_copy(acc_ref.at[pl.ds(sid * STRIPE, WB_FULL)],
                                out_hbm.at[pl.ds(o_base, WB_FULL)])

            @pl.when(sid == NS - 1)
            def _():
                pltpu.sync_copy(acc_ref.at[pl.ds(sid * STRIPE, WB_LAST)],
                                out_hbm.at[pl.ds(o_base, WB_LAST)])

        zero_acc_a()
        for t in range(STRIPE // 224):
            pltpu.sync_copy(zbuf_b, acc_b.at[pl.ds(sid * STRIPE + t * 224, 224)])
        pltpu.sync_copy(zcnt, cnt.at[pl.ds(sid * STRIPE, STRIPE)])
        plsc.subcore_barrier()
        edge_loop(with_b=True)
        plsc.subcore_barrier()
        writeback(acc_a, sa_hbm)
        writeback(acc_b, sb_hbm)
        writeback(cnt, cnt_hbm)

        zero_acc_a()
        plsc.subcore_barrier()
        edge_loop(with_b=False)
        plsc.subcore_barrier()
        writeback(acc_a, sc_hbm)

    return k(src, dst, ph_a, ph_b, ph_c)


BN = 1000


def _tc_body(x_ref, cnt_ref, sa_ref, sb_ref, sc_ref, smb_ref, w_ref, r_ref,
             rt_ref, lik_ref, post_ref):
    inv = 1.0 / jnp.maximum(cnt_ref[...], 1.0)
    sums = jnp.concatenate([sa_ref[...], sc_ref[...], sb_ref[...]], axis=1)
    aggr = sums * inv
    s1 = jnp.dot(aggr, w_ref[...], preferred_element_type=jnp.float32)
    iota = lax.broadcasted_iota(jnp.int32, (BN, M), 1)
    oh = (iota == x_ref[...]).astype(jnp.float32)
    bn = jnp.dot(oh, smb_ref[...], preferred_element_type=jnp.float32)
    t = bn * s1
    d = jnp.dot(t, r_ref[...], preferred_element_type=jnp.float32) + 1e-6
    lik_ref[...] = jnp.log(d)
    dexp = jnp.dot(1.0 / d, rt_ref[...], preferred_element_type=jnp.float32)
    post_ref[...] = (t + 1e-7) * dexp


def _tc_dense(x2, cnt2, sa, sb, sc, smb_flat, w, r, rt):
    grid = (N // BN,)
    return pl.pallas_call(
        _tc_body,
        grid=grid,
        in_specs=[
            pl.BlockSpec((BN, 1), lambda i: (i, 0)),
            pl.BlockSpec((BN, 1), lambda i: (i, 0)),
            pl.BlockSpec((BN, FA), lambda i: (i, 0)),
            pl.BlockSpec((BN, FB), lambda i: (i, 0)),
            pl.BlockSpec((BN, FA), lambda i: (i, 0)),
            pl.BlockSpec((M, F), lambda i: (0, 0)),
            pl.BlockSpec((F, F), lambda i: (0, 0)),
            pl.BlockSpec((F, NG), lambda i: (0, 0)),
            pl.BlockSpec((NG, F), lambda i: (0, 0)),
        ],
        out_specs=[
            pl.BlockSpec((BN, NG), lambda i: (i, 0)),
            pl.BlockSpec((BN, F), lambda i: (i, 0)),
        ],
        out_shape=[
            jax.ShapeDtypeStruct((N, NG), jnp.float32),
            jax.ShapeDtypeStruct((N, F), jnp.float32),
        ],
    )(x2, cnt2, sa, sb, sc, smb_flat, w, r, rt)


def kernel(x, prev_h, edge_index, Q_neigh, B):
    sm_Q = jax.nn.softmax(Q_neigh, axis=0)
    sm_B = jax.nn.softmax(B, axis=1)
    eye = jnp.eye(NG, dtype=jnp.float32)
    w = jnp.einsum("ilg,gh->lgih", sm_Q, eye).reshape(F, F)
    smb_flat = jnp.transpose(sm_B, (1, 0, 2)).reshape(M, F)
    r = jnp.tile(eye, (C, 1))
    rt = r.T

    src = edge_index[0].astype(jnp.int32)
    dst = edge_index[1].astype(jnp.int32)
    pad = E_PAD - E
    src = jnp.concatenate([src, jnp.full((pad,), N, jnp.int32)])
    dst = jnp.concatenate([dst, jnp.zeros((pad,), jnp.int32)])
    prevh_flat = prev_h.reshape(N, F)
    ph_a = prevh_flat[:, :FA]
    ph_c = prevh_flat[:, FA:2 * FA]
    ph_b = prevh_flat[:, 2 * FA:]

    sa, sb, sc, counts = _sc_segment_sum(src, dst, ph_a, ph_b, ph_c)

    x2 = x.astype(jnp.int32).reshape(N, 1)
    cnt2 = counts.reshape(N, 1)
    lik, post = _tc_dense(x2, cnt2, sa, sb, sc, smb_flat, w, r, rt)
    return lik, post.reshape(N, C, NG)


# speedup vs baseline: 75.8175x; 1.3762x over previous
import functools

import jax
import jax.numpy as jnp
from jax import lax
from jax.experimental import pallas as pl
from jax.experimental.pallas import tpu as pltpu
from jax.experimental.pallas import tpu_sc as plsc

N = 50000
E = 800000
C = 10
M = 32
NG = 8
F = C * NG
FA = 32
FB = 16

NC = 2
NS = 16
LANES = 16

HALF = N // NC
ROWS = 25088
DUMP = HALF
STRIPE = ROWS // NS
WB_FULL = STRIPE
WB_LAST = HALF - 15 * STRIPE

KR = 128
SCH = 2
RPT = 392
SUPER = RPT // SCH
EPT = RPT * KR
E_PAD = NS * EPT


def _sc_segment_sum(src2d, dst2d, ph_a, ph_b, ph_c):
    mesh = plsc.VectorSubcoreMesh(core_axis_name="c", subcore_axis_name="s",
                                  num_cores=NC, num_subcores=NS)

    @functools.partial(
        pl.kernel,
        out_type=(
            jax.ShapeDtypeStruct((N, FA), jnp.float32),
            jax.ShapeDtypeStruct((N, FB), jnp.float32),
            jax.ShapeDtypeStruct((N, FA), jnp.float32),
            jax.ShapeDtypeStruct((N,), jnp.float32),
        ),
        mesh=mesh,
        compiler_params=pltpu.CompilerParams(use_tc_tiling_on_sc=False),
        scratch_types=dict(
            src_v=[pltpu.VMEM((SCH, KR), jnp.int32) for _ in range(2)],
            dst_v=[pltpu.VMEM((SCH, KR), jnp.int32) for _ in range(2)],
            idx_v=[pltpu.VMEM((SCH, KR), jnp.int32) for _ in range(2)],
            ones_v=pltpu.VMEM((KR,), jnp.float32),
            rows_a=[pltpu.VMEM((SCH * KR, FA), jnp.float32) for _ in range(2)],
            rows_b=[pltpu.VMEM((SCH * KR, FB), jnp.float32) for _ in range(2)],
            zcnt=pltpu.VMEM((STRIPE,), jnp.float32),
            acc_a=pltpu.VMEM_SHARED((ROWS, FA), jnp.float32),
            acc_b=pltpu.VMEM_SHARED((ROWS, FB), jnp.float32),
            cnt=pltpu.VMEM_SHARED((ROWS,), jnp.float32),
            sem_a=[pltpu.SemaphoreType.DMA for _ in range(2)],
            sem_b=[pltpu.SemaphoreType.DMA for _ in range(2)],
        ),
    )
    def k(src_hbm, dst_hbm, a_hbm, b_hbm, c_hbm, sa_hbm, sb_hbm, sc_hbm,
          cnt_hbm, *, src_v, dst_v, idx_v, ones_v, rows_a, rows_b,
          zcnt, acc_a, acc_b, cnt, sem_a, sem_b):
        cid = lax.axis_index("c")
        sid = lax.axis_index("s")
        base = cid * HALF
        o_base = cid * HALF + sid * STRIPE

        zero16 = jnp.zeros((LANES,), jnp.float32)
        one16 = jnp.ones((LANES,), jnp.float32)
        for j in range(KR // LANES):
            ones_v[pl.ds(j * LANES, LANES)] = one16

        def zero_rows(nrows):
            def zrow(i, carry):
                for j in range(FA // LANES):
                    rows_a[0][i, pl.ds(j * LANES, LANES)] = zero16
                rows_b[0][i, pl.ds(0, LANES)] = zero16
                return carry
            lax.fori_loop(0, nrows, zrow, 0)

        def zc(i, carry):
            zcnt[pl.ds(i * LANES, LANES)] = zero16
            return carry
        lax.fori_loop(0, STRIPE // LANES, zc, 0)

        def zero_acc(ref, zsrc):
            for t in range(STRIPE // 224):
                pltpu.sync_copy(zsrc,
                                ref.at[pl.ds(sid * STRIPE + t * 224, 224)])

        def edge_loop(with_b):
            tbl_a = a_hbm if with_b else c_hbm

            def load(p, j):
                row0 = sid * RPT + j * SCH
                pltpu.sync_copy(src_hbm.at[pl.ds(row0, SCH)], src_v[p])
                pltpu.sync_copy(dst_hbm.at[pl.ds(row0, SCH)], dst_v[p])

            def fire(p):
                for i in range(SCH):
                    pltpu.async_copy(tbl_a.at[dst_v[p].at[i]],
                                     rows_a[p].at[pl.ds(i * KR, KR)], sem_a[p])
                    if with_b:
                        pltpu.async_copy(b_hbm.at[dst_v[p].at[i]],
                                         rows_b[p].at[pl.ds(i * KR, KR)],
                                         sem_b[p])

            def drain(p):
                for i in range(SCH):
                    pltpu.make_async_copy(
                        tbl_a.at[dst_v[p].at[i]],
                        rows_a[p].at[pl.ds(i * KR, KR)], sem_a[p]).wait()
                    if with_b:
                        pltpu.make_async_copy(
                            b_hbm.at[dst_v[p].at[i]],
                            rows_b[p].at[pl.ds(i * KR, KR)], sem_b[p]).wait()

            def scatter(p):
                for i in range(SCH):
                    for q in range(KR // LANES):
                        s16 = src_v[p][i, pl.ds(q * LANES, LANES)]
                        loc = s16 - base
                        ok = (loc >= 0) & (loc < HALF)
                        idx_v[p][i, pl.ds(q * LANES, LANES)] = (
                            jnp.where(ok, loc, DUMP))
                for i in range(SCH):
                    pltpu.sync_copy(rows_a[p].at[pl.ds(i * KR, KR)],
                                    acc_a.at[idx_v[p].at[i]], add=True)
                    if with_b:
                        pltpu.sync_copy(rows_b[p].at[pl.ds(i * KR, KR)],
                                        acc_b.at[idx_v[p].at[i]], add=True)
                        pltpu.sync_copy(ones_v, cnt.at[idx_v[p].at[i]],
                                        add=True)

            for p in range(2):
                load(p, p)
                fire(p)

            def body(step, carry):
                for p in range(2):
                    j = step * 2 + p
                    drain(p)
                    scatter(p)
                    load(p, j + 2)
                    fire(p)
                return carry
            lax.fori_loop(0, SUPER // 2 - 1, body, 0)
            for p in range(2):
                drain(p)
                scatter(p)

        def writeback(acc_ref, out_hbm, width_is_b=False):
            @pl.when(sid < NS - 1)
            def _():
                pltpu.sync_copy(acc_ref.at[pl.ds(sid * STRIPE, WB_FULL)],
                                out_hbm.at[pl.ds(o_base, WB_FULL)])

            @pl.when(sid == NS - 1)
            def _():
                pltpu.sync_copy(acc_ref.at[pl.ds(sid * STRIPE, WB_LAST)],
                                out_hbm.at[pl.ds(o_base, WB_LAST)])

        zero_rows(224)
        zero_acc(acc_a, rows_a[0].at[pl.ds(0, 224)])
        zero_acc(acc_b, rows_b[0].at[pl.ds(0, 224)])
        pltpu.sync_copy(zcnt, cnt.at[pl.ds(sid * STRIPE, STRIPE)])
        plsc.subcore_barrier()
        edge_loop(with_b=True)
        plsc.subcore_barrier()
        writeback(acc_a, sa_hbm)
        writeback(acc_b, sb_hbm)
        writeback(cnt, cnt_hbm)

        zero_rows(224)
        zero_acc(acc_a, rows_a[0].at[pl.ds(0, 224)])
        plsc.subcore_barrier()
        edge_loop(with_b=False)
        plsc.subcore_barrier()
        writeback(acc_a, sc_hbm)

    return k(src2d, dst2d, ph_a, ph_b, ph_c)


BN = 1000


def _tc_body(x_ref, cnt_ref, sa_ref, sb_ref, sc_ref, smb_ref, w_ref, r_ref,
             rt_ref, lik_ref, post_ref):
    inv = 1.0 / jnp.maximum(cnt_ref[...], 1.0)
    sums = jnp.concatenate([sa_ref[...], sc_ref[...], sb_ref[...]], axis=1)
    aggr = sums * inv
    s1 = jnp.dot(aggr, w_ref[...], preferred_element_type=jnp.float32)
    iota = lax.broadcasted_iota(jnp.int32, (BN, M), 1)
    oh = (iota == x_ref[...]).astype(jnp.float32)
    bn = jnp.dot(oh, smb_ref[...], preferred_element_type=jnp.float32)
    t = bn * s1
    d = jnp.dot(t, r_ref[...], preferred_element_type=jnp.float32) + 1e-6
    lik_ref[...] = jnp.log(d)
    dexp = jnp.dot(1.0 / d, rt_ref[...], preferred_element_type=jnp.float32)
    post_ref[...] = (t + 1e-7) * dexp


def _tc_dense(x2, cnt2, sa, sb, sc, smb_flat, w, r, rt):
    grid = (N // BN,)
    return pl.pallas_call(
        _tc_body,
        grid=grid,
        in_specs=[
            pl.BlockSpec((BN, 1), lambda i: (i, 0)),
            pl.BlockSpec((BN, 1), lambda i: (i, 0)),
            pl.BlockSpec((BN, FA), lambda i: (i, 0)),
            pl.BlockSpec((BN, FB), lambda i: (i, 0)),
            pl.BlockSpec((BN, FA), lambda i: (i, 0)),
            pl.BlockSpec((M, F), lambda i: (0, 0)),
            pl.BlockSpec((F, F), lambda i: (0, 0)),
            pl.BlockSpec((F, NG), lambda i: (0, 0)),
            pl.BlockSpec((NG, F), lambda i: (0, 0)),
        ],
        out_specs=[
            pl.BlockSpec((BN, NG), lambda i: (i, 0)),
            pl.BlockSpec((BN, F), lambda i: (i, 0)),
        ],
        out_shape=[
            jax.ShapeDtypeStruct((N, NG), jnp.float32),
            jax.ShapeDtypeStruct((N, F), jnp.float32),
        ],
    )(x2, cnt2, sa, sb, sc, smb_flat, w, r, rt)


def kernel(x, prev_h, edge_index, Q_neigh, B):
    sm_Q = jax.nn.softmax(Q_neigh, axis=0)
    sm_B = jax.nn.softmax(B, axis=1)
    eye = jnp.eye(NG, dtype=jnp.float32)
    w = jnp.einsum("ilg,gh->lgih", sm_Q, eye).reshape(F, F)
    smb_flat = jnp.transpose(sm_B, (1, 0, 2)).reshape(M, F)
    r = jnp.tile(eye, (C, 1))
    rt = r.T

    src = edge_index[0].astype(jnp.int32)
    dst = edge_index[1].astype(jnp.int32)
    pad = E_PAD - E
    src = jnp.concatenate([src, jnp.full((pad,), N, jnp.int32)]).reshape(-1, KR)
    dst = jnp.concatenate([dst, jnp.zeros((pad,), jnp.int32)]).reshape(-1, KR)
    prevh_flat = prev_h.reshape(N, F)
    ph_a = prevh_flat[:, :FA]
    ph_c = prevh_flat[:, FA:2 * FA]
    ph_b = prevh_flat[:, 2 * FA:]

    sa, sb, sc, counts = _sc_segment_sum(src, dst, ph_a, ph_b, ph_c)

    x2 = x.astype(jnp.int32).reshape(N, 1)
    cnt2 = counts.reshape(N, 1)
    lik, post = _tc_dense(x2, cnt2, sa, sb, sc, smb_flat, w, r, rt)
    return lik, post.reshape(N, C, NG)


# speedup vs baseline: 77.0702x; 1.0165x over previous
import functools

import jax
import jax.numpy as jnp
from jax import lax
from jax.experimental import pallas as pl
from jax.experimental.pallas import tpu as pltpu
from jax.experimental.pallas import tpu_sc as plsc

N = 50000
E = 800000
C = 10
M = 32
NG = 8
F = C * NG
FA = 32
FB = 16

NC = 2
NS = 16
LANES = 16

HALF = N // NC
ROWS = 25088
DUMP = HALF
STRIPE = ROWS // NS
WB_FULL = STRIPE
WB_LAST = HALF - 15 * STRIPE

KR = 128
SCH = 2
RPT = 392
SUPER = RPT // SCH
EPT = RPT * KR
E_PAD = NS * EPT


def _sc_segment_sum(src2d, dst2d, ph_a, ph_b, ph_c):
    mesh = plsc.VectorSubcoreMesh(core_axis_name="c", subcore_axis_name="s",
                                  num_cores=NC, num_subcores=NS)

    @functools.partial(
        pl.kernel,
        out_type=(
            jax.ShapeDtypeStruct((N, FA), jnp.float32),
            jax.ShapeDtypeStruct((N, FB), jnp.float32),
            jax.ShapeDtypeStruct((N, FA), jnp.float32),
            jax.ShapeDtypeStruct((N,), jnp.float32),
        ),
        mesh=mesh,
        compiler_params=pltpu.CompilerParams(use_tc_tiling_on_sc=False),
        scratch_types=dict(
            src_v=[pltpu.VMEM((SCH, KR), jnp.int32) for _ in range(2)],
            dst_v=[pltpu.VMEM((SCH, KR), jnp.int32) for _ in range(2)],
            idx_v=[pltpu.VMEM((SCH, KR), jnp.int32) for _ in range(2)],
            ones_v=pltpu.VMEM((KR,), jnp.float32),
            rows_a=[pltpu.VMEM((SCH * KR, FA), jnp.float32) for _ in range(2)],
            rows_b=[pltpu.VMEM((SCH * KR, FB), jnp.float32) for _ in range(2)],
            zcnt=pltpu.VMEM((STRIPE,), jnp.float32),
            acc_a=pltpu.VMEM_SHARED((ROWS, FA), jnp.float32),
            acc_b=pltpu.VMEM_SHARED((ROWS, FB), jnp.float32),
            cnt=pltpu.VMEM_SHARED((ROWS,), jnp.float32),
            sem_a=[pltpu.SemaphoreType.DMA for _ in range(2)],
            sem_b=[pltpu.SemaphoreType.DMA for _ in range(2)],
            sem_l=[pltpu.SemaphoreType.DMA for _ in range(2)],
            sem_s=[pltpu.SemaphoreType.DMA for _ in range(2)],
        ),
    )
    def k(src_hbm, dst_hbm, a_hbm, b_hbm, c_hbm, sa_hbm, sb_hbm, sc_hbm,
          cnt_hbm, *, src_v, dst_v, idx_v, ones_v, rows_a, rows_b,
          zcnt, acc_a, acc_b, cnt, sem_a, sem_b, sem_l, sem_s):
        cid = lax.axis_index("c")
        sid = lax.axis_index("s")
        base = cid * HALF
        o_base = cid * HALF + sid * STRIPE

        zero16 = jnp.zeros((LANES,), jnp.float32)
        one16 = jnp.ones((LANES,), jnp.float32)
        for j in range(KR // LANES):
            ones_v[pl.ds(j * LANES, LANES)] = one16

        def zero_rows(nrows):
            def zrow(i, carry):
                for j in range(FA // LANES):
                    rows_a[0][i, pl.ds(j * LANES, LANES)] = zero16
                rows_b[0][i, pl.ds(0, LANES)] = zero16
                return carry
            lax.fori_loop(0, nrows, zrow, 0)

        def zc(i, carry):
            zcnt[pl.ds(i * LANES, LANES)] = zero16
            return carry
        lax.fori_loop(0, STRIPE // LANES, zc, 0)

        def zero_acc(ref, zsrc):
            for t in range(STRIPE // 224):
                pltpu.sync_copy(zsrc,
                                ref.at[pl.ds(sid * STRIPE + t * 224, 224)])

        def edge_loop(with_b):
            tbl_a = a_hbm if with_b else c_hbm

            def fire_loads(p, j):
                row0 = sid * RPT + j * SCH
                pltpu.async_copy(src_hbm.at[pl.ds(row0, SCH)], src_v[p],
                                 sem_l[p])
                pltpu.async_copy(dst_hbm.at[pl.ds(row0, SCH)], dst_v[p],
                                 sem_l[p])

            def drain_loads(p):
                pltpu.make_async_copy(src_hbm.at[pl.ds(0, SCH)], src_v[p],
                                      sem_l[p]).wait()
                pltpu.make_async_copy(dst_hbm.at[pl.ds(0, SCH)], dst_v[p],
                                      sem_l[p]).wait()

            def fire_gathers(p):
                for i in range(SCH):
                    pltpu.async_copy(tbl_a.at[dst_v[p].at[i]],
                                     rows_a[p].at[pl.ds(i * KR, KR)], sem_a[p])
                    if with_b:
                        pltpu.async_copy(b_hbm.at[dst_v[p].at[i]],
                                         rows_b[p].at[pl.ds(i * KR, KR)],
                                         sem_b[p])

            def drain_gathers(p):
                for i in range(SCH):
                    pltpu.make_async_copy(
                        tbl_a.at[dst_v[p].at[i]],
                        rows_a[p].at[pl.ds(i * KR, KR)], sem_a[p]).wait()
                    if with_b:
                        pltpu.make_async_copy(
                            b_hbm.at[dst_v[p].at[i]],
                            rows_b[p].at[pl.ds(i * KR, KR)], sem_b[p]).wait()

            def compute_idx(p):
                for i in range(SCH):
                    for q in range(KR // LANES):
                        s16 = src_v[p][i, pl.ds(q * LANES, LANES)]
                        loc = s16 - base
                        ok = (loc >= 0) & (loc < HALF)
                        idx_v[p][i, pl.ds(q * LANES, LANES)] = (
                            jnp.where(ok, loc, DUMP))

            def fire_scatters(p):
                for i in range(SCH):
                    pltpu.async_copy(rows_a[p].at[pl.ds(i * KR, KR)],
                                     acc_a.at[idx_v[p].at[i]], sem_s[p],
                                     add=True)
                    if with_b:
                        pltpu.async_copy(rows_b[p].at[pl.ds(i * KR, KR)],
                                         acc_b.at[idx_v[p].at[i]], sem_s[p],
                                         add=True)
                        pltpu.async_copy(ones_v, cnt.at[idx_v[p].at[i]],
                                        sem_s[p], add=True)

            def drain_scatters(p):
                for i in range(SCH):
                    pltpu.make_async_copy(rows_a[p].at[pl.ds(i * KR, KR)],
                                          acc_a.at[idx_v[p].at[i]],
                                          sem_s[p]).wait()
                    if with_b:
                        pltpu.make_async_copy(rows_b[p].at[pl.ds(i * KR, KR)],
                                              acc_b.at[idx_v[p].at[i]],
                                              sem_s[p]).wait()
                        pltpu.make_async_copy(ones_v, cnt.at[idx_v[p].at[i]],
                                              sem_s[p]).wait()

            for p in range(2):
                row0 = sid * RPT + p * SCH
                pltpu.sync_copy(src_hbm.at[pl.ds(row0, SCH)], src_v[p])
                pltpu.sync_copy(dst_hbm.at[pl.ds(row0, SCH)], dst_v[p])
                fire_gathers(p)

            def body(step, carry):
                for p in range(2):
                    j = step * 2 + p
                    drain_gathers(p)
                    compute_idx(p)
                    fire_loads(p, j + 2)
                    fire_scatters(p)
                    drain_scatters(p)
                    drain_loads(p)
                    fire_gathers(p)
                return carry
            lax.fori_loop(0, SUPER // 2 - 1, body, 0)
            for p in range(2):
                drain_gathers(p)
                compute_idx(p)
                fire_scatters(p)
                drain_scatters(p)

        def writeback(acc_ref, out_hbm, width_is_b=False):
            @pl.when(sid < NS - 1)
            def _():
                pltpu.sync_copy(acc_ref.at[pl.ds(sid * STRIPE, WB_FULL)],
                                out_hbm.at[pl.ds(o_base, WB_FULL)])

            @pl.when(sid == NS - 1)
            def _():
                pltpu.sync_copy(acc_ref.at[pl.ds(sid * STRIPE, WB_LAST)],
                                out_hbm.at[pl.ds(o_base, WB_LAST)])

        zero_rows(224)
        zero_acc(acc_a, rows_a[0].at[pl.ds(0, 224)])
        zero_acc(acc_b, rows_b[0].at[pl.ds(0, 224)])
        pltpu.sync_copy(zcnt, cnt.at[pl.ds(sid * STRIPE, STRIPE)])
        plsc.subcore_barrier()
        edge_loop(with_b=True)
        plsc.subcore_barrier()
        writeback(acc_a, sa_hbm)
        writeback(acc_b, sb_hbm)
        writeback(cnt, cnt_hbm)

        zero_rows(224)
        zero_acc(acc_a, rows_a[0].at[pl.ds(0, 224)])
        plsc.subcore_barrier()
        edge_loop(with_b=False)
        plsc.subcore_barrier()
        writeback(acc_a, sc_hbm)

    return k(src2d, dst2d, ph_a, ph_b, ph_c)


BN = 1000


def _tc_body(x_ref, cnt_ref, sa_ref, sb_ref, sc_ref, smb_ref, w_ref, r_ref,
             rt_ref, lik_ref, post_ref):
    inv = 1.0 / jnp.maximum(cnt_ref[...], 1.0)
    sums = jnp.concatenate([sa_ref[...], sc_ref[...], sb_ref[...]], axis=1)
    aggr = sums * inv
    s1 = jnp.dot(aggr, w_ref[...], preferred_element_type=jnp.float32)
    iota = lax.broadcasted_iota(jnp.int32, (BN, M), 1)
    oh = (iota == x_ref[...]).astype(jnp.float32)
    bn = jnp.dot(oh, smb_ref[...], preferred_element_type=jnp.float32)
    t = bn * s1
    d = jnp.dot(t, r_ref[...], preferred_element_type=jnp.float32) + 1e-6
    lik_ref[...] = jnp.log(d)
    dexp = jnp.dot(1.0 / d, rt_ref[...], preferred_element_type=jnp.float32)
    post_ref[...] = (t + 1e-7) * dexp


def _tc_dense(x2, cnt2, sa, sb, sc, smb_flat, w, r, rt):
    grid = (N // BN,)
    return pl.pallas_call(
        _tc_body,
        grid=grid,
        in_specs=[
            pl.BlockSpec((BN, 1), lambda i: (i, 0)),
            pl.BlockSpec((BN, 1), lambda i: (i, 0)),
            pl.BlockSpec((BN, FA), lambda i: (i, 0)),
            pl.BlockSpec((BN, FB), lambda i: (i, 0)),
            pl.BlockSpec((BN, FA), lambda i: (i, 0)),
            pl.BlockSpec((M, F), lambda i: (0, 0)),
            pl.BlockSpec((F, F), lambda i: (0, 0)),
            pl.BlockSpec((F, NG), lambda i: (0, 0)),
            pl.BlockSpec((NG, F), lambda i: (0, 0)),
        ],
        out_specs=[
            pl.BlockSpec((BN, NG), lambda i: (i, 0)),
            pl.BlockSpec((BN, F), lambda i: (i, 0)),
        ],
        out_shape=[
            jax.ShapeDtypeStruct((N, NG), jnp.float32),
            jax.ShapeDtypeStruct((N, F), jnp.float32),
        ],
    )(x2, cnt2, sa, sb, sc, smb_flat, w, r, rt)


def kernel(x, prev_h, edge_index, Q_neigh, B):
    sm_Q = jax.nn.softmax(Q_neigh, axis=0)
    sm_B = jax.nn.softmax(B, axis=1)
    eye = jnp.eye(NG, dtype=jnp.float32)
    w = jnp.einsum("ilg,gh->lgih", sm_Q, eye).reshape(F, F)
    smb_flat = jnp.transpose(sm_B, (1, 0, 2)).reshape(M, F)
    r = jnp.tile(eye, (C, 1))
    rt = r.T

    src = edge_index[0].astype(jnp.int32)
    dst = edge_index[1].astype(jnp.int32)
    pad = E_PAD - E
    src = jnp.concatenate([src, jnp.full((pad,), N, jnp.int32)]).reshape(-1, KR)
    dst = jnp.concatenate([dst, jnp.zeros((pad,), jnp.int32)]).reshape(-1, KR)
    prevh_flat = prev_h.reshape(N, F)
    ph_a = prevh_flat[:, :FA]
    ph_c = prevh_flat[:, FA:2 * FA]
    ph_b = prevh_flat[:, 2 * FA:]

    sa, sb, sc, counts = _sc_segment_sum(src, dst, ph_a, ph_b, ph_c)

    x2 = x.astype(jnp.int32).reshape(N, 1)
    cnt2 = counts.reshape(N, 1)
    lik, post = _tc_dense(x2, cnt2, sa, sb, sc, smb_flat, w, r, rt)
    return lik, post.reshape(N, C, NG)


# speedup vs baseline: 77.1280x; 1.0008x over previous
import functools

import jax
import jax.numpy as jnp
from jax import lax
from jax.experimental import pallas as pl
from jax.experimental.pallas import tpu as pltpu
from jax.experimental.pallas import tpu_sc as plsc

N = 50000
E = 800000
C = 10
M = 32
NG = 8
F = C * NG
FA = 32
FB = 16

NC = 2
NS = 16
LANES = 16

HALF = N // NC
ROWS = 25088
DUMP = HALF
STRIPE = ROWS // NS
WB_FULL = STRIPE
WB_LAST = HALF - 15 * STRIPE

KR = 128
SCH = 4
RPT = 392
SUPER = RPT // SCH
EPT = RPT * KR
E_PAD = NS * EPT


def _sc_segment_sum(src2d, dst2d, ph_a, ph_b, ph_c):
    mesh = plsc.VectorSubcoreMesh(core_axis_name="c", subcore_axis_name="s",
                                  num_cores=NC, num_subcores=NS)

    @functools.partial(
        pl.kernel,
        out_type=(
            jax.ShapeDtypeStruct((N, FA), jnp.float32),
            jax.ShapeDtypeStruct((N, FB), jnp.float32),
            jax.ShapeDtypeStruct((N, FA), jnp.float32),
            jax.ShapeDtypeStruct((N,), jnp.float32),
        ),
        mesh=mesh,
        compiler_params=pltpu.CompilerParams(use_tc_tiling_on_sc=False),
        scratch_types=dict(
            src_v=[pltpu.VMEM((SCH, KR), jnp.int32) for _ in range(2)],
            dst_v=[pltpu.VMEM((SCH, KR), jnp.int32) for _ in range(2)],
            idx_v=[pltpu.VMEM((SCH, KR), jnp.int32) for _ in range(2)],
            ones_v=pltpu.VMEM((KR,), jnp.float32),
            rows_a=[pltpu.VMEM((SCH * KR, FA), jnp.float32) for _ in range(2)],
            rows_b=[pltpu.VMEM((SCH * KR, FB), jnp.float32) for _ in range(2)],
            zcnt=pltpu.VMEM((STRIPE // 4,), jnp.float32),
            acc_a=pltpu.VMEM_SHARED((ROWS, FA), jnp.float32),
            acc_b=pltpu.VMEM_SHARED((ROWS, FB), jnp.float32),
            cnt=pltpu.VMEM_SHARED((ROWS,), jnp.float32),
            sem_a=[pltpu.SemaphoreType.DMA for _ in range(2)],
            sem_b=[pltpu.SemaphoreType.DMA for _ in range(2)],
            sem_l=[pltpu.SemaphoreType.DMA for _ in range(2)],
            sem_s=[pltpu.SemaphoreType.DMA for _ in range(2)],
        ),
    )
    def k(src_hbm, dst_hbm, a_hbm, b_hbm, c_hbm, sa_hbm, sb_hbm, sc_hbm,
          cnt_hbm, *, src_v, dst_v, idx_v, ones_v, rows_a, rows_b,
          zcnt, acc_a, acc_b, cnt, sem_a, sem_b, sem_l, sem_s):
        cid = lax.axis_index("c")
        sid = lax.axis_index("s")
        base = cid * HALF
        o_base = cid * HALF + sid * STRIPE

        zero16 = jnp.zeros((LANES,), jnp.float32)
        one16 = jnp.ones((LANES,), jnp.float32)
        for j in range(KR // LANES):
            ones_v[pl.ds(j * LANES, LANES)] = one16

        def zero_rows(nrows):
            def zrow(i, carry):
                for j in range(FA // LANES):
                    rows_a[0][i, pl.ds(j * LANES, LANES)] = zero16
                rows_b[0][i, pl.ds(0, LANES)] = zero16
                return carry
            lax.fori_loop(0, nrows, zrow, 0)

        def zc(i, carry):
            zcnt[pl.ds(i * LANES, LANES)] = zero16
            return carry
        lax.fori_loop(0, STRIPE // 4 // LANES, zc, 0)

        def zero_acc(ref, zsrc):
            for t in range(STRIPE // 224):
                pltpu.sync_copy(zsrc,
                                ref.at[pl.ds(sid * STRIPE + t * 224, 224)])

        def edge_loop(with_b):
            tbl_a = a_hbm if with_b else c_hbm

            def fire_loads(p, j):
                row0 = sid * RPT + j * SCH
                pltpu.async_copy(src_hbm.at[pl.ds(row0, SCH)], src_v[p],
                                 sem_l[p])
                pltpu.async_copy(dst_hbm.at[pl.ds(row0, SCH)], dst_v[p],
                                 sem_l[p])

            def drain_loads(p):
                pltpu.make_async_copy(src_hbm.at[pl.ds(0, SCH)], src_v[p],
                                      sem_l[p]).wait()
                pltpu.make_async_copy(dst_hbm.at[pl.ds(0, SCH)], dst_v[p],
                                      sem_l[p]).wait()

            def fire_gathers(p):
                for i in range(SCH):
                    pltpu.async_copy(tbl_a.at[dst_v[p].at[i]],
                                     rows_a[p].at[pl.ds(i * KR, KR)], sem_a[p])
                    if with_b:
                        pltpu.async_copy(b_hbm.at[dst_v[p].at[i]],
                                         rows_b[p].at[pl.ds(i * KR, KR)],
                                         sem_b[p])

            def drain_gathers(p):
                for i in range(SCH):
                    pltpu.make_async_copy(
                        tbl_a.at[dst_v[p].at[i]],
                        rows_a[p].at[pl.ds(i * KR, KR)], sem_a[p]).wait()
                    if with_b:
                        pltpu.make_async_copy(
                            b_hbm.at[dst_v[p].at[i]],
                            rows_b[p].at[pl.ds(i * KR, KR)], sem_b[p]).wait()

            def compute_idx(p):
                for i in range(SCH):
                    for q in range(KR // LANES):
                        s16 = src_v[p][i, pl.ds(q * LANES, LANES)]
                        loc = s16 - base
                        ok = (loc >= 0) & (loc < HALF)
                        idx_v[p][i, pl.ds(q * LANES, LANES)] = (
                            jnp.where(ok, loc, DUMP))

            def fire_scatters(p):
                for i in range(SCH):
                    pltpu.async_copy(rows_a[p].at[pl.ds(i * KR, KR)],
                                     acc_a.at[idx_v[p].at[i]], sem_s[p],
                                     add=True)
                    if with_b:
                        pltpu.async_copy(rows_b[p].at[pl.ds(i * KR, KR)],
                                         acc_b.at[idx_v[p].at[i]], sem_s[p],
                                         add=True)
                        pltpu.async_copy(ones_v, cnt.at[idx_v[p].at[i]],
                                        sem_s[p], add=True)

            def drain_scatters(p):
                for i in range(SCH):
                    pltpu.make_async_copy(rows_a[p].at[pl.ds(i * KR, KR)],
                                          acc_a.at[idx_v[p].at[i]],
                                          sem_s[p]).wait()
                    if with_b:
                        pltpu.make_async_copy(rows_b[p].at[pl.ds(i * KR, KR)],
                                              acc_b.at[idx_v[p].at[i]],
                                              sem_s[p]).wait()
                        pltpu.make_async_copy(ones_v, cnt.at[idx_v[p].at[i]],
                                              sem_s[p]).wait()

            for p in range(2):
                row0 = sid * RPT + p * SCH
                pltpu.sync_copy(src_hbm.at[pl.ds(row0, SCH)], src_v[p])
                pltpu.sync_copy(dst_hbm.at[pl.ds(row0, SCH)], dst_v[p])
                fire_gathers(p)

            def body(step, carry):
                for p in range(2):
                    j = step * 2 + p
                    drain_gathers(p)
                    compute_idx(p)
                    fire_loads(p, j + 2)
                    fire_scatters(p)
                    drain_scatters(p)
                    drain_loads(p)
                    fire_gathers(p)
                return carry
            lax.fori_loop(0, SUPER // 2 - 1, body, 0)
            for p in range(2):
                drain_gathers(p)
                compute_idx(p)
                fire_scatters(p)
                drain_scatters(p)

        def writeback(acc_ref, out_hbm, width_is_b=False):
            @pl.when(sid < NS - 1)
            def _():
                pltpu.sync_copy(acc_ref.at[pl.ds(sid * STRIPE, WB_FULL)],
                                out_hbm.at[pl.ds(o_base, WB_FULL)])

            @pl.when(sid == NS - 1)
            def _():
                pltpu.sync_copy(acc_ref.at[pl.ds(sid * STRIPE, WB_LAST)],
                                out_hbm.at[pl.ds(o_base, WB_LAST)])

        zero_rows(224)
        zero_acc(acc_a, rows_a[0].at[pl.ds(0, 224)])
        zero_acc(acc_b, rows_b[0].at[pl.ds(0, 224)])
        for t4 in range(4):
            pltpu.sync_copy(zcnt, cnt.at[pl.ds(sid * STRIPE + t4 * (STRIPE // 4), STRIPE // 4)])
        plsc.subcore_barrier()
        edge_loop(with_b=True)
        plsc.subcore_barrier()
        writeback(acc_a, sa_hbm)
        writeback(acc_b, sb_hbm)
        writeback(cnt, cnt_hbm)

        zero_rows(224)
        zero_acc(acc_a, rows_a[0].at[pl.ds(0, 224)])
        plsc.subcore_barrier()
        edge_loop(with_b=False)
        plsc.subcore_barrier()
        writeback(acc_a, sc_hbm)

    return k(src2d, dst2d, ph_a, ph_b, ph_c)


BN = 1000


def _tc_body(x_ref, cnt_ref, sa_ref, sb_ref, sc_ref, smb_ref, w_ref, r_ref,
             rt_ref, lik_ref, post_ref):
    inv = 1.0 / jnp.maximum(cnt_ref[...], 1.0)
    sums = jnp.concatenate([sa_ref[...], sc_ref[...], sb_ref[...]], axis=1)
    aggr = sums * inv
    s1 = jnp.dot(aggr, w_ref[...], preferred_element_type=jnp.float32)
    iota = lax.broadcasted_iota(jnp.int32, (BN, M), 1)
    oh = (iota == x_ref[...]).astype(jnp.float32)
    bn = jnp.dot(oh, smb_ref[...], preferred_element_type=jnp.float32)
    t = bn * s1
    d = jnp.dot(t, r_ref[...], preferred_element_type=jnp.float32) + 1e-6
    lik_ref[...] = jnp.log(d)
    dexp = jnp.dot(1.0 / d, rt_ref[...], preferred_element_type=jnp.float32)
    post_ref[...] = (t + 1e-7) * dexp


def _tc_dense(x2, cnt2, sa, sb, sc, smb_flat, w, r, rt):
    grid = (N // BN,)
    return pl.pallas_call(
        _tc_body,
        grid=grid,
        in_specs=[
            pl.BlockSpec((BN, 1), lambda i: (i, 0)),
            pl.BlockSpec((BN, 1), lambda i: (i, 0)),
            pl.BlockSpec((BN, FA), lambda i: (i, 0)),
            pl.BlockSpec((BN, FB), lambda i: (i, 0)),
            pl.BlockSpec((BN, FA), lambda i: (i, 0)),
            pl.BlockSpec((M, F), lambda i: (0, 0)),
            pl.BlockSpec((F, F), lambda i: (0, 0)),
            pl.BlockSpec((F, NG), lambda i: (0, 0)),
            pl.BlockSpec((NG, F), lambda i: (0, 0)),
        ],
        out_specs=[
            pl.BlockSpec((BN, NG), lambda i: (i, 0)),
            pl.BlockSpec((BN, F), lambda i: (i, 0)),
        ],
        out_shape=[
            jax.ShapeDtypeStruct((N, NG), jnp.float32),
            jax.ShapeDtypeStruct((N, F), jnp.float32),
        ],
    )(x2, cnt2, sa, sb, sc, smb_flat, w, r, rt)


def kernel(x, prev_h, edge_index, Q_neigh, B):
    sm_Q = jax.nn.softmax(Q_neigh, axis=0)
    sm_B = jax.nn.softmax(B, axis=1)
    eye = jnp.eye(NG, dtype=jnp.float32)
    w = jnp.einsum("ilg,gh->lgih", sm_Q, eye).reshape(F, F)
    smb_flat = jnp.transpose(sm_B, (1, 0, 2)).reshape(M, F)
    r = jnp.tile(eye, (C, 1))
    rt = r.T

    src = edge_index[0].astype(jnp.int32)
    dst = edge_index[1].astype(jnp.int32)
    pad = E_PAD - E
    src = jnp.concatenate([src, jnp.full((pad,), N, jnp.int32)]).reshape(-1, KR)
    dst = jnp.concatenate([dst, jnp.zeros((pad,), jnp.int32)]).reshape(-1, KR)
    prevh_flat = prev_h.reshape(N, F)
    ph_a = prevh_flat[:, :FA]
    ph_c = prevh_flat[:, FA:2 * FA]
    ph_b = prevh_flat[:, 2 * FA:]

    sa, sb, sc, counts = _sc_segment_sum(src, dst, ph_a, ph_b, ph_c)

    x2 = x.astype(jnp.int32).reshape(N, 1)
    cnt2 = counts.reshape(N, 1)
    lik, post = _tc_dense(x2, cnt2, sa, sb, sc, smb_flat, w, r, rt)
    return lik, post.reshape(N, C, NG)
